# Initial kernel scaffold; baseline (speedup 1.0000x reference)
import jax
import jax.numpy as jnp
from jax.experimental import pallas as pl


def kernel(node_feat, edge_index, phi1, W1, b1, phi_hidden, phi2, W2, b2):
    raise NotImplementedError("write your pallas kernel here")


# speedup vs baseline: 6.5117x; 6.5117x over previous
import functools

import jax
import jax.numpy as jnp
from jax import lax
from jax.experimental import pallas as pl
from jax.experimental.pallas import tpu as pltpu
from jax.experimental.pallas import tpu_sc as plsc

N = 10000
NP = 10240
D = 128
E = 320000
CH = 128
NC = 2
NS = 16
NW = NC * NS
KCH = (-((E + NW * CH - 1) // (NW * CH)) // 8) * -8
EP = NW * KCH * CH
RPS = NP // NS

_mesh = plsc.VectorSubcoreMesh(core_axis_name="c", subcore_axis_name="s")


@functools.partial(
    pl.kernel,
    out_type=(
        jax.ShapeDtypeStruct((NC, NP), jnp.float32),
        jax.ShapeDtypeStruct((NC, NP), jnp.float32),
    ),
    mesh=_mesh,
    scratch_types=[
        pltpu.VMEM((KCH, CH), jnp.int32),
        pltpu.VMEM((KCH, CH), jnp.int32),
        pltpu.VMEM((CH,), jnp.float32),
        pltpu.VMEM((1, CH), jnp.int32),
        pltpu.VMEM_SHARED((NP,), jnp.float32),
        pltpu.VMEM_SHARED((NP,), jnp.float32),
    ],
)
def _sc_hist(row_hbm, col_hbm, zeros1_hbm, degp, selfp,
             rows_v, cols_v, ones_v, sel_v, deg_sh, self_sh):
    c = lax.axis_index("c")
    s = lax.axis_index("s")
    w = s * NC + c
    pltpu.sync_copy(zeros1_hbm.at[pl.ds(s * RPS, RPS)],
                    deg_sh.at[pl.ds(s * RPS, RPS)])
    pltpu.sync_copy(zeros1_hbm.at[pl.ds(s * RPS, RPS)],
                    self_sh.at[pl.ds(s * RPS, RPS)])
    for j in range(CH // 16):
        ones_v[pl.ds(j * 16, 16)] = jnp.ones((16,), jnp.float32)
    pltpu.sync_copy(row_hbm.at[pl.ds(w * KCH, KCH)], rows_v)
    pltpu.sync_copy(col_hbm.at[pl.ds(w * KCH, KCH)], cols_v)
    plsc.subcore_barrier()

    @pl.loop(0, KCH)
    def _(k):
        pltpu.sync_copy(ones_v, deg_sh.at[cols_v.at[k]], add=True)
        for j in range(CH // 16):
            r = rows_v[k, pl.ds(j * 16, 16)]
            cc = cols_v[k, pl.ds(j * 16, 16)]
            sel_v[0, pl.ds(j * 16, 16)] = jnp.where(r == cc, cc, NP - 1)
        pltpu.sync_copy(ones_v, self_sh.at[sel_v.at[0]], add=True)

    plsc.subcore_barrier()
    pltpu.sync_copy(deg_sh.at[pl.ds(s * RPS, RPS)],
                    degp.at[c, pl.ds(s * RPS, RPS)])
    pltpu.sync_copy(self_sh.at[pl.ds(s * RPS, RPS)],
                    selfp.at[c, pl.ds(s * RPS, RPS)])


HALF = KCH // 2


@functools.partial(
    pl.kernel,
    out_type=jax.ShapeDtypeStruct((NC, NP, D), jnp.float32),
    mesh=_mesh,
    scratch_types=[
        pltpu.VMEM((HALF, CH), jnp.int32),
        pltpu.VMEM((HALF, CH), jnp.int32),
        pltpu.VMEM((CH, D), jnp.float32),
        pltpu.VMEM((CH, D), jnp.float32),
        pltpu.VMEM_SHARED((NP, D), jnp.float32),
        pltpu.SemaphoreType.DMA,
        pltpu.SemaphoreType.DMA,
        pltpu.SemaphoreType.DMA,
        pltpu.SemaphoreType.DMA,
    ],
)
def _sc_gs(xs_hbm, row_hbm, col_hbm, zeros2_hbm, gp,
           rows_v, cols_v, buf0, buf1, g_sh, sg0, sg1, ss0, ss1):
    c = lax.axis_index("c")
    s = lax.axis_index("s")
    w = s * NC + c
    pltpu.sync_copy(zeros2_hbm.at[pl.ds(s * RPS, RPS)],
                    g_sh.at[pl.ds(s * RPS, RPS)])
    plsc.subcore_barrier()

    for h in range(2):
        pltpu.sync_copy(row_hbm.at[pl.ds(w * KCH + h * HALF, HALF)], rows_v)
        pltpu.sync_copy(col_hbm.at[pl.ds(w * KCH + h * HALF, HALF)], cols_v)

        pltpu.async_copy(xs_hbm.at[cols_v.at[0]], buf0, sg0)

        @pl.loop(0, HALF - 1)
        def _(k):
            even = lax.rem(k, 2) == 0

            @pl.when(even)
            def _():
                pltpu.async_copy(xs_hbm.at[cols_v.at[k + 1]], buf1, sg1)
                pltpu.make_async_copy(xs_hbm.at[cols_v.at[k]],
                                      buf0, sg0).wait()
                pltpu.async_copy(buf0, g_sh.at[rows_v.at[k]],
                                 ss0, add=True).wait()

            @pl.when(jnp.logical_not(even))
            def _():
                pltpu.async_copy(xs_hbm.at[cols_v.at[k + 1]], buf0, sg0)
                pltpu.make_async_copy(xs_hbm.at[cols_v.at[k + 1]],
                                      buf1, sg1).wait()
                pltpu.async_copy(buf1, g_sh.at[rows_v.at[k]],
                                 ss1, add=True).wait()

        last = HALF - 1
        lbuf, lsem = (buf0, sg0) if last % 2 == 0 else (buf1, sg1)
        pltpu.make_async_copy(xs_hbm.at[cols_v.at[last]], lbuf, lsem).wait()
        pltpu.async_copy(lbuf, g_sh.at[rows_v.at[last]], ss0, add=True).wait()

    plsc.subcore_barrier()
    pltpu.sync_copy(g_sh.at[pl.ds(s * RPS, RPS)],
                    gp.at[c, pl.ds(s * RPS, RPS)])


_R = 1024


def _prep_body(d0, d1, s0, s1, dis_ref, u_ref):
    deg = d0[...] + d1[...] + 1.0
    cnt = s0[...] + s1[...]
    i = (lax.broadcasted_iota(jnp.int32, (NP // 128, 128), 0) * 128
         + lax.broadcasted_iota(jnp.int32, (NP // 128, 128), 1))
    mask = i < N
    dis_ref[...] = jnp.where(mask, lax.rsqrt(deg), 0.0)
    u_ref[...] = jnp.where(mask, (deg - 1.0) / deg + cnt, 0.0)


def _tc_prep(d0, d1, s0, s1):
    f = pl.pallas_call(
        _prep_body,
        out_shape=(jax.ShapeDtypeStruct((NP // 128, 128), jnp.float32),
                   jax.ShapeDtypeStruct((NP // 128, 128), jnp.float32)),
    )
    return f(d0, d1, s0, s1)


def _scale_body(x, dis, xs_ref):
    xs_ref[...] = x[...] * dis[...]


def _tc_scale(x, dis):
    f = pl.pallas_call(
        _scale_body,
        grid=(NP // _R,),
        in_specs=[
            pl.BlockSpec((_R, D), lambda i: (i, 0)),
            pl.BlockSpec((_R, 1), lambda i: (i, 0)),
        ],
        out_specs=pl.BlockSpec((_R, D), lambda i: (i, 0)),
        out_shape=jax.ShapeDtypeStruct((NP, D), jnp.float32),
    )
    return f(x, dis)


def _mid_body(x, g0, g1, dis, u, phi, y_ref, ys_ref):
    g = g0[...] + g1[...]
    sp = u[...] * x[...] - dis[...] * g
    y = x[...] - phi[...] * sp
    y_ref[...] = y
    ys_ref[...] = dis[...] * y


def _tc_mid(x, g0, g1, dis, u, phi):
    f = pl.pallas_call(
        _mid_body,
        grid=(NP // _R,),
        in_specs=[
            pl.BlockSpec((_R, D), lambda i: (i, 0)),
            pl.BlockSpec((_R, D), lambda i: (i, 0)),
            pl.BlockSpec((_R, D), lambda i: (i, 0)),
            pl.BlockSpec((_R, 1), lambda i: (i, 0)),
            pl.BlockSpec((_R, 1), lambda i: (i, 0)),
            pl.BlockSpec((1, D), lambda i: (0, 0)),
        ],
        out_specs=(pl.BlockSpec((_R, D), lambda i: (i, 0)),
                   pl.BlockSpec((_R, D), lambda i: (i, 0))),
        out_shape=(jax.ShapeDtypeStruct((NP, D), jnp.float32),
                   jax.ShapeDtypeStruct((NP, D), jnp.float32)),
    )
    return f(x, g0, g1, dis, u, phi)


def _mm_body(relu, x, g0, g1, dis, u, phi, W, b, y_ref, ys_ref=None):
    g = g0[...] + g1[...]
    z = x[...] - phi[...] * (u[...] * x[...] - dis[...] * g)
    y = jnp.dot(z, W[...], preferred_element_type=jnp.float32) + b[...]
    if relu:
        y = jnp.maximum(y, 0.0)
    y_ref[...] = y
    if ys_ref is not None:
        ys_ref[...] = dis[...] * y


def _tc_mm(x, g0, g1, dis, u, phi, W, b, relu, want_ys):
    nout = 2 if want_ys else 1
    blk = pl.BlockSpec((_R, D), lambda i: (i, 0))
    out_specs = (blk, blk) if want_ys else blk
    out_shape = tuple(jax.ShapeDtypeStruct((NP, D), jnp.float32)
                      for _ in range(nout))
    if not want_ys:
        out_shape = out_shape[0]
    f = pl.pallas_call(
        functools.partial(_mm_body, relu),
        grid=(NP // _R,),
        in_specs=[
            blk, blk, blk,
            pl.BlockSpec((_R, 1), lambda i: (i, 0)),
            pl.BlockSpec((_R, 1), lambda i: (i, 0)),
            pl.BlockSpec((1, D), lambda i: (0, 0)),
            pl.BlockSpec((D, D), lambda i: (0, 0)),
            pl.BlockSpec((1, D), lambda i: (0, 0)),
        ],
        out_specs=out_specs,
        out_shape=out_shape,
    )
    return f(x, g0, g1, dis, u, phi, W, b)


@jax.jit
def _run(node_feat, edge_index, phi1, W1, b1, phi_hidden, phi2, W2, b2):
    xpad = jnp.pad(node_feat, ((0, NP - N), (0, 0)))
    rowp = jnp.pad(edge_index[0], (0, EP - E),
                   constant_values=NP - 1).reshape(NW * KCH, CH)
    colp = jnp.pad(edge_index[1], (0, EP - E),
                   constant_values=NP - 1).reshape(NW * KCH, CH)
    zeros1 = jnp.zeros((NP,), jnp.float32)
    zeros2 = jnp.zeros((NP, D), jnp.float32)

    degp, selfp = _sc_hist(rowp, colp, zeros1)
    dis2d, u2d = _tc_prep(degp[0].reshape(NP // 128, 128),
                          degp[1].reshape(NP // 128, 128),
                          selfp[0].reshape(NP // 128, 128),
                          selfp[1].reshape(NP // 128, 128))
    dis = dis2d.reshape(NP, 1)
    u = u2d.reshape(NP, 1)

    xs = _tc_scale(xpad, dis)
    g = _sc_gs(xs, rowp, colp, zeros2)
    x1, xs = _tc_mm(xpad, g[0], g[1], dis, u, phi1.reshape(1, D), W1,
                    b1.reshape(1, D), relu=True, want_ys=True)
    g = _sc_gs(xs, rowp, colp, zeros2)
    x2, xs = _tc_mid(x1, g[0], g[1], dis, u, phi_hidden[0].reshape(1, D))
    g = _sc_gs(xs, rowp, colp, zeros2)
    x3, xs = _tc_mid(x2, g[0], g[1], dis, u, phi_hidden[1].reshape(1, D))
    g = _sc_gs(xs, rowp, colp, zeros2)
    out = _tc_mm(x3, g[0], g[1], dis, u, phi2.reshape(1, D), W2,
                 b2.reshape(1, D), relu=False, want_ys=False)
    return out[:N]


def kernel(node_feat, edge_index, phi1, W1, b1, phi_hidden, phi2, W2, b2):
    return _run(node_feat, edge_index, phi1, W1, b1, phi_hidden, phi2, W2, b2)
